# baseline (speedup 1.0000x reference)
import jax
import jax.numpy as jnp
import numpy as np
import scipy.special as _sp
from jax import lax
from jax.experimental import pallas as pl
from jax.experimental.pallas import tpu as pltpu

_S = 2048
_D = 768
_E = 8
_KH = _D
_T = 256
_DC = 128
_NC = _D // _DC
_NT = _S // _T
_BN = _E * _DC

def _threefry2x32_np(k0, k1, x0, x1):
    rot_even = (13, 15, 26, 6)
    rot_odd = (17, 29, 16, 24)

    def _rotl(x, r):
        return ((x << np.uint32(r)) | (x >> np.uint32(32 - r))).astype(np.uint32)

    ks = (np.uint32(k0), np.uint32(k1),
          np.uint32(np.uint32(k0) ^ np.uint32(k1) ^ np.uint32(0x1BD11BDA)))
    x0 = (x0 + ks[0]).astype(np.uint32)
    x1 = (x1 + ks[1]).astype(np.uint32)
    for d in range(5):
        for r in rot_even if d % 2 == 0 else rot_odd:
            x0 = (x0 + x1).astype(np.uint32)
            x1 = _rotl(x1, r)
            x1 = (x1 ^ x0).astype(np.uint32)
        x0 = (x0 + ks[(d + 1) % 3]).astype(np.uint32)
        x1 = (x1 + ks[(d + 2) % 3] + np.uint32(d + 1)).astype(np.uint32)
    return x0, x1


def _noise_natural_np():
    n = _S * _D * _E
    i = np.arange(n, dtype=np.uint64)
    hi32 = (i >> np.uint64(32)).astype(np.uint32)
    lo32 = (i & np.uint64(0xFFFFFFFF)).astype(np.uint32)
    b0, b1 = _threefry2x32_np(0, 12345, hi32, lo32)
    bits = b0 ^ b1
    mant = (bits >> np.uint32(9)) | np.float32(1.0).view(np.uint32)
    f = mant.view(np.float32) - np.float32(1.0)
    lo_f = np.float32(np.nextafter(np.float32(-1.0), np.float32(0.0)))
    u = np.maximum(lo_f, (f * (np.float32(1.0) - lo_f) + lo_f).astype(np.float32))
    norm = (np.float32(np.sqrt(2.0))
            * _sp.erfinv(u.astype(np.float64)).astype(np.float32))
    return norm.astype(np.float32).reshape(_S, _D * _E)


_NOISE = _noise_natural_np()

_PDE = np.zeros((_BN, _BN), dtype=np.float32)
_i = np.arange(_BN)
_PDE[_i, (_i % _E) * _DC + _i // _E] = 1.0


def _bias_kernel(hcat_ref, wr_ref, br_ref, wnn_ref, wno_ref, we_ref,
                 bnn_ref, bno_ref, be_ref, onn_ref, ono_ref, oe_ref, r8):
    @pl.when(pl.program_id(0) == 0)
    def _():
        r8[...] = (
            jnp.dot(hcat_ref[...], wr_ref[...], preferred_element_type=jnp.float32)
            + br_ref[...]
        )

    r = r8[...]
    onn_ref[...] = jnp.dot(r, wnn_ref[...], preferred_element_type=jnp.float32) + bnn_ref[...]
    ono_ref[...] = jnp.dot(r, wno_ref[...], preferred_element_type=jnp.float32) + bno_ref[...]
    oe_ref[...] = jnp.dot(r, we_ref[...], preferred_element_type=jnp.float32) + be_ref[...]


def _main_kernel(x_ref, wnnl_ref, wnnh_ref, wnol_ref, wnoh_ref, wel_ref, weh_ref,
                 bnn_ref, bno_ref, be_ref, nz_ref, pde_ref, out_ref):
    f32 = jnp.float32
    xl = x_ref[:, :_KH]
    xh = x_ref[:, _KH:]
    y_nn = (jnp.dot(xl, wnnl_ref[...], preferred_element_type=f32)
            + jnp.dot(xh, wnnh_ref[...], preferred_element_type=f32)
            + bnn_ref[0][None, :])
    y_no = (jnp.dot(xl, wnol_ref[...], preferred_element_type=f32)
            + jnp.dot(xh, wnoh_ref[...], preferred_element_type=f32)
            + bno_ref[0][None, :])
    y_e = (jnp.dot(xl, wel_ref[...], preferred_element_type=f32)
           + jnp.dot(xh, weh_ref[...], preferred_element_type=f32)
           + be_ref[0][None, :])
    hs = y_nn + y_no * nz_ref[...]

    pos = lax.broadcasted_iota(jnp.int32, hs.shape, 1) % _E
    b = lax.bitcast_convert_type(hs, jnp.int32)
    o = b ^ (lax.shift_right_arithmetic(b, 31) & jnp.int32(0x7FFFFFFF))
    kint = (o & jnp.int32(~7)) | (jnp.int32(_E - 1) - pos)

    bf16 = jnp.bfloat16
    pde = pde_ref[...]
    cs_de = []
    for sh in (24, 16, 8, 0):
        c = lax.shift_right_arithmetic(kint, sh)
        if sh != 24:
            c = c & jnp.int32(0xFF)
        c_de = jnp.dot(lax.convert_element_type(c, bf16), pde,
                       preferred_element_type=f32)
        cs_de.append(lax.convert_element_type(c_de, jnp.int32))
    k_de = (lax.shift_left(cs_de[0], 24) | lax.shift_left(cs_de[1], 16)
            | lax.shift_left(cs_de[2], 8) | cs_de[3])
    ye_hi = lax.convert_element_type(y_e, bf16)
    ye_lo = lax.convert_element_type(
        y_e - lax.convert_element_type(ye_hi, f32), bf16)
    ye_de = (jnp.dot(ye_hi, pde, preferred_element_type=f32)
             + jnp.dot(ye_lo, pde, preferred_element_type=f32))

    ks = [k_de[:, e * _DC:(e + 1) * _DC] for e in range(_E)]
    m1k = ks[0]
    for e in range(1, _E):
        m1k = jnp.maximum(m1k, ks[e])
    k2s = [jnp.where(k == m1k, jnp.int32(-(2**31)), k) for k in ks]
    m2k = k2s[0]
    for e in range(1, _E):
        m2k = jnp.maximum(m2k, k2s[e])

    def _to_f32(v):
        return lax.bitcast_convert_type(
            v ^ (lax.shift_right_arithmetic(v, 31) & jnp.int32(0x7FFFFFFF)),
            jnp.float32)

    s = jnp.exp(_to_f32(m2k) - _to_f32(m1k))
    inv_z = 1.0 / (1.0 + s)
    g2 = s * inv_z
    acc = None
    for e in range(_E):
        ge = jnp.where(ks[e] == m1k, inv_z,
                       jnp.where(k2s[e] == m2k, g2, 0.0))
        term = ge * ye_de[:, e * _DC:(e + 1) * _DC]
        acc = term if acc is None else acc + term
    out_ref[...] = acc * (1.0 / _E)


def kernel(h, us, ue, u, W_non_noise, b_non_noise, W_noise, b_noise, W_E, b_E, W_r, b_r):
    f32 = jnp.float32

    hcat8 = jnp.broadcast_to(
        jnp.concatenate([h, us, ue], axis=-1).reshape(1, 5 * _D), (8, 5 * _D)
    )
    br8 = jnp.broadcast_to(b_r[None, :], (8, _D))
    bnn8 = jnp.broadcast_to(b_non_noise[None, :], (8, _D * _E))
    bno8 = jnp.broadcast_to(b_noise[None, :], (8, _D * _E))
    be8 = jnp.broadcast_to(b_E[None, :], (8, _D * _E))
    x2d = u.reshape(_S, 2 * _D)

    row2 = pl.BlockSpec((_KH, _BN), lambda c: (2, c))
    bspec = pl.BlockSpec((8, _BN), lambda c: (0, c))
    beff_nn, beff_no, beff_e = pl.pallas_call(
        _bias_kernel,
        grid=(_NC,),
        in_specs=[
            pl.BlockSpec((8, 5 * _D), lambda c: (0, 0)),
            pl.BlockSpec((5 * _D, _D), lambda c: (0, 0)),
            pl.BlockSpec((8, _D), lambda c: (0, 0)),
            row2, row2, row2,
            bspec, bspec, bspec,
        ],
        out_specs=[bspec, bspec, bspec],
        out_shape=[jax.ShapeDtypeStruct((8, _D * _E), f32)] * 3,
        scratch_shapes=[pltpu.VMEM((8, _D), f32)],
    )(hcat8, W_r, br8, W_non_noise, W_noise, W_E, bnn8, bno8, be8)

    row0 = pl.BlockSpec((_KH, _BN), lambda c, t: (0, c))
    row1 = pl.BlockSpec((_KH, _BN), lambda c, t: (1, c))
    bspec2 = pl.BlockSpec((8, _BN), lambda c, t: (0, c))
    out2d = pl.pallas_call(
        _main_kernel,
        grid=(_NC, _NT),
        in_specs=[
            pl.BlockSpec((_T, 2 * _D), lambda c, t: (t, 0)),
            row0, row1, row0, row1, row0, row1,
            bspec2, bspec2, bspec2,
            pl.BlockSpec((_T, _BN), lambda c, t: (t, c)),
            pl.BlockSpec((_BN, _BN), lambda c, t: (0, 0)),
        ],
        out_specs=pl.BlockSpec((_T, _DC), lambda c, t: (t, c)),
        out_shape=jax.ShapeDtypeStruct((_S, _D), f32),
    )(x2d, W_non_noise, W_non_noise, W_noise, W_noise, W_E, W_E,
      beff_nn, beff_no, beff_e, jnp.asarray(_NOISE),
      jnp.asarray(_PDE, jnp.bfloat16))

    return out2d.reshape(1, _S, _D)


# speedup vs baseline: 1.0699x; 1.0699x over previous
import jax
import jax.numpy as jnp
import numpy as np
import scipy.special as _sp
from jax import lax
from jax.experimental import pallas as pl
from jax.experimental.pallas import tpu as pltpu

_S = 2048
_D = 768
_E = 8
_KH = _D
_T = 512
_DC = 128
_NC = _D // _DC
_NT = _S // _T
_BN = _E * _DC

def _threefry2x32_np(k0, k1, x0, x1):
    rot_even = (13, 15, 26, 6)
    rot_odd = (17, 29, 16, 24)

    def _rotl(x, r):
        return ((x << np.uint32(r)) | (x >> np.uint32(32 - r))).astype(np.uint32)

    ks = (np.uint32(k0), np.uint32(k1),
          np.uint32(np.uint32(k0) ^ np.uint32(k1) ^ np.uint32(0x1BD11BDA)))
    x0 = (x0 + ks[0]).astype(np.uint32)
    x1 = (x1 + ks[1]).astype(np.uint32)
    for d in range(5):
        for r in rot_even if d % 2 == 0 else rot_odd:
            x0 = (x0 + x1).astype(np.uint32)
            x1 = _rotl(x1, r)
            x1 = (x1 ^ x0).astype(np.uint32)
        x0 = (x0 + ks[(d + 1) % 3]).astype(np.uint32)
        x1 = (x1 + ks[(d + 2) % 3] + np.uint32(d + 1)).astype(np.uint32)
    return x0, x1


def _noise_natural_np():
    n = _S * _D * _E
    i = np.arange(n, dtype=np.uint64)
    hi32 = (i >> np.uint64(32)).astype(np.uint32)
    lo32 = (i & np.uint64(0xFFFFFFFF)).astype(np.uint32)
    b0, b1 = _threefry2x32_np(0, 12345, hi32, lo32)
    bits = b0 ^ b1
    mant = (bits >> np.uint32(9)) | np.float32(1.0).view(np.uint32)
    f = mant.view(np.float32) - np.float32(1.0)
    lo_f = np.float32(np.nextafter(np.float32(-1.0), np.float32(0.0)))
    u = np.maximum(lo_f, (f * (np.float32(1.0) - lo_f) + lo_f).astype(np.float32))
    norm = (np.float32(np.sqrt(2.0))
            * _sp.erfinv(u.astype(np.float64)).astype(np.float32))
    return norm.astype(np.float32).reshape(_S, _D * _E)


_NOISE = _noise_natural_np()

_PDE = np.zeros((_BN, _BN), dtype=np.float32)
_i = np.arange(_BN)
_PDE[_i, (_i % _E) * _DC + _i // _E] = 1.0


def _bias_kernel(hcat_ref, wr_ref, br_ref, wnn_ref, wno_ref, we_ref,
                 bnn_ref, bno_ref, be_ref, onn_ref, ono_ref, oe_ref, r8):
    @pl.when(pl.program_id(0) == 0)
    def _():
        r8[...] = (
            jnp.dot(hcat_ref[...], wr_ref[...], preferred_element_type=jnp.float32)
            + br_ref[...]
        )

    r = r8[...]
    onn_ref[...] = jnp.dot(r, wnn_ref[...], preferred_element_type=jnp.float32) + bnn_ref[...]
    ono_ref[...] = jnp.dot(r, wno_ref[...], preferred_element_type=jnp.float32) + bno_ref[...]
    oe_ref[...] = jnp.dot(r, we_ref[...], preferred_element_type=jnp.float32) + be_ref[...]


def _main_kernel(x_ref, wnnl_ref, wnnh_ref, wnol_ref, wnoh_ref, wel_ref, weh_ref,
                 bnn_ref, bno_ref, be_ref, nz_ref, pde_ref, out_ref):
    f32 = jnp.float32
    xl = x_ref[:, :_KH]
    xh = x_ref[:, _KH:]
    y_nn = (jnp.dot(xl, wnnl_ref[...], preferred_element_type=f32)
            + jnp.dot(xh, wnnh_ref[...], preferred_element_type=f32)
            + bnn_ref[0][None, :])
    y_no = (jnp.dot(xl, wnol_ref[...], preferred_element_type=f32)
            + jnp.dot(xh, wnoh_ref[...], preferred_element_type=f32)
            + bno_ref[0][None, :])
    y_e = (jnp.dot(xl, wel_ref[...], preferred_element_type=f32)
           + jnp.dot(xh, weh_ref[...], preferred_element_type=f32)
           + be_ref[0][None, :])
    hs = y_nn + y_no * nz_ref[...]

    pos = lax.broadcasted_iota(jnp.int32, hs.shape, 1) % _E
    b = lax.bitcast_convert_type(hs, jnp.int32)
    o = b ^ (lax.shift_right_arithmetic(b, 31) & jnp.int32(0x7FFFFFFF))
    kint = (o & jnp.int32(~7)) | (jnp.int32(_E - 1) - pos)

    bf16 = jnp.bfloat16
    pde = pde_ref[...]
    cs_de = []
    for sh in (24, 16, 8, 0):
        c = lax.shift_right_arithmetic(kint, sh)
        if sh != 24:
            c = c & jnp.int32(0xFF)
        c_de = jnp.dot(lax.convert_element_type(c, bf16), pde,
                       preferred_element_type=f32)
        cs_de.append(lax.convert_element_type(c_de, jnp.int32))
    k_de = (lax.shift_left(cs_de[0], 24) | lax.shift_left(cs_de[1], 16)
            | lax.shift_left(cs_de[2], 8) | cs_de[3])
    ye_hi = lax.convert_element_type(y_e, bf16)
    ye_lo = lax.convert_element_type(
        y_e - lax.convert_element_type(ye_hi, f32), bf16)
    ye_de = (jnp.dot(ye_hi, pde, preferred_element_type=f32)
             + jnp.dot(ye_lo, pde, preferred_element_type=f32))

    ks = [k_de[:, e * _DC:(e + 1) * _DC] for e in range(_E)]
    m1k = ks[0]
    for e in range(1, _E):
        m1k = jnp.maximum(m1k, ks[e])
    k2s = [jnp.where(k == m1k, jnp.int32(-(2**31)), k) for k in ks]
    m2k = k2s[0]
    for e in range(1, _E):
        m2k = jnp.maximum(m2k, k2s[e])

    def _to_f32(v):
        return lax.bitcast_convert_type(
            v ^ (lax.shift_right_arithmetic(v, 31) & jnp.int32(0x7FFFFFFF)),
            jnp.float32)

    s = jnp.exp(_to_f32(m2k) - _to_f32(m1k))
    inv_z = 1.0 / (1.0 + s)
    g2 = s * inv_z
    acc = None
    for e in range(_E):
        ge = jnp.where(ks[e] == m1k, inv_z,
                       jnp.where(k2s[e] == m2k, g2, 0.0))
        term = ge * ye_de[:, e * _DC:(e + 1) * _DC]
        acc = term if acc is None else acc + term
    out_ref[...] = acc * (1.0 / _E)


def kernel(h, us, ue, u, W_non_noise, b_non_noise, W_noise, b_noise, W_E, b_E, W_r, b_r):
    f32 = jnp.float32

    hcat8 = jnp.broadcast_to(
        jnp.concatenate([h, us, ue], axis=-1).reshape(1, 5 * _D), (8, 5 * _D)
    )
    br8 = jnp.broadcast_to(b_r[None, :], (8, _D))
    bnn8 = jnp.broadcast_to(b_non_noise[None, :], (8, _D * _E))
    bno8 = jnp.broadcast_to(b_noise[None, :], (8, _D * _E))
    be8 = jnp.broadcast_to(b_E[None, :], (8, _D * _E))
    x2d = u.reshape(_S, 2 * _D)

    row2 = pl.BlockSpec((_KH, _BN), lambda c: (2, c))
    bspec = pl.BlockSpec((8, _BN), lambda c: (0, c))
    beff_nn, beff_no, beff_e = pl.pallas_call(
        _bias_kernel,
        grid=(_NC,),
        in_specs=[
            pl.BlockSpec((8, 5 * _D), lambda c: (0, 0)),
            pl.BlockSpec((5 * _D, _D), lambda c: (0, 0)),
            pl.BlockSpec((8, _D), lambda c: (0, 0)),
            row2, row2, row2,
            bspec, bspec, bspec,
        ],
        out_specs=[bspec, bspec, bspec],
        out_shape=[jax.ShapeDtypeStruct((8, _D * _E), f32)] * 3,
        scratch_shapes=[pltpu.VMEM((8, _D), f32)],
    )(hcat8, W_r, br8, W_non_noise, W_noise, W_E, bnn8, bno8, be8)

    row0 = pl.BlockSpec((_KH, _BN), lambda c, t: (0, c))
    row1 = pl.BlockSpec((_KH, _BN), lambda c, t: (1, c))
    bspec2 = pl.BlockSpec((8, _BN), lambda c, t: (0, c))
    out2d = pl.pallas_call(
        _main_kernel,
        grid=(_NC, _NT),
        in_specs=[
            pl.BlockSpec((_T, 2 * _D), lambda c, t: (t, 0)),
            row0, row1, row0, row1, row0, row1,
            bspec2, bspec2, bspec2,
            pl.BlockSpec((_T, _BN), lambda c, t: (t, c)),
            pl.BlockSpec((_BN, _BN), lambda c, t: (0, 0)),
        ],
        out_specs=pl.BlockSpec((_T, _DC), lambda c, t: (t, c)),
        out_shape=jax.ShapeDtypeStruct((_S, _D), f32),
    )(x2d, W_non_noise, W_non_noise, W_noise, W_noise, W_E, W_E,
      beff_nn, beff_no, beff_e, jnp.asarray(_NOISE),
      jnp.asarray(_PDE, jnp.bfloat16))

    return out2d.reshape(1, _S, _D)


# speedup vs baseline: 1.1648x; 1.0887x over previous
import jax
import jax.numpy as jnp
import numpy as np
import scipy.special as _sp
from jax import lax
from jax.experimental import pallas as pl
from jax.experimental.pallas import tpu as pltpu

_S = 2048
_D = 768
_E = 8
_KH = _D
_T = 512
_DC = 128
_NC = _D // _DC
_NT = _S // _T
_BN = _E * _DC

def _threefry2x32_np(k0, k1, x0, x1):
    rot_even = (13, 15, 26, 6)
    rot_odd = (17, 29, 16, 24)

    def _rotl(x, r):
        return ((x << np.uint32(r)) | (x >> np.uint32(32 - r))).astype(np.uint32)

    ks = (np.uint32(k0), np.uint32(k1),
          np.uint32(np.uint32(k0) ^ np.uint32(k1) ^ np.uint32(0x1BD11BDA)))
    x0 = (x0 + ks[0]).astype(np.uint32)
    x1 = (x1 + ks[1]).astype(np.uint32)
    for d in range(5):
        for r in rot_even if d % 2 == 0 else rot_odd:
            x0 = (x0 + x1).astype(np.uint32)
            x1 = _rotl(x1, r)
            x1 = (x1 ^ x0).astype(np.uint32)
        x0 = (x0 + ks[(d + 1) % 3]).astype(np.uint32)
        x1 = (x1 + ks[(d + 2) % 3] + np.uint32(d + 1)).astype(np.uint32)
    return x0, x1


def _noise_natural_np():
    n = _S * _D * _E
    i = np.arange(n, dtype=np.uint64)
    hi32 = (i >> np.uint64(32)).astype(np.uint32)
    lo32 = (i & np.uint64(0xFFFFFFFF)).astype(np.uint32)
    b0, b1 = _threefry2x32_np(0, 12345, hi32, lo32)
    bits = b0 ^ b1
    mant = (bits >> np.uint32(9)) | np.float32(1.0).view(np.uint32)
    f = mant.view(np.float32) - np.float32(1.0)
    lo_f = np.float32(np.nextafter(np.float32(-1.0), np.float32(0.0)))
    u = np.maximum(lo_f, (f * (np.float32(1.0) - lo_f) + lo_f).astype(np.float32))
    norm = (np.float32(np.sqrt(2.0))
            * _sp.erfinv(u.astype(np.float64)).astype(np.float32))
    return norm.astype(np.float32).reshape(_S, _D * _E)


_NOISE = _noise_natural_np()

_PDE = np.zeros((_BN, _BN), dtype=np.float32)
_i = np.arange(_BN)
_PDE[_i, (_i % _E) * _DC + _i // _E] = 1.0


def _bias_kernel(hcat_ref, wr_ref, br_ref, wnn_ref, wno_ref, we_ref,
                 bnn_ref, bno_ref, be_ref, onn_ref, ono_ref, oe_ref, r8):
    @pl.when(pl.program_id(0) == 0)
    def _():
        r8[...] = (
            jnp.dot(hcat_ref[...], wr_ref[...], preferred_element_type=jnp.float32)
            + br_ref[...]
        )

    r = r8[...]
    onn_ref[...] = jnp.dot(r, wnn_ref[...], preferred_element_type=jnp.float32) + bnn_ref[...]
    ono_ref[...] = jnp.dot(r, wno_ref[...], preferred_element_type=jnp.float32) + bno_ref[...]
    oe_ref[...] = jnp.dot(r, we_ref[...], preferred_element_type=jnp.float32) + be_ref[...]


def _main_kernel(x_ref, wnnl_ref, wnnh_ref, wnol_ref, wnoh_ref, wel_ref, weh_ref,
                 bnn_ref, bno_ref, be_ref, nz_ref, pde_ref, out_ref):
    f32 = jnp.float32
    xl = x_ref[:, :_KH]
    xh = x_ref[:, _KH:]
    y_nn = (jnp.dot(xl, wnnl_ref[...], preferred_element_type=f32)
            + jnp.dot(xh, wnnh_ref[...], preferred_element_type=f32)
            + bnn_ref[0][None, :])
    y_no = (jnp.dot(xl, wnol_ref[...], preferred_element_type=f32)
            + jnp.dot(xh, wnoh_ref[...], preferred_element_type=f32)
            + bno_ref[0][None, :])
    y_e = (jnp.dot(xl, wel_ref[...], preferred_element_type=f32)
           + jnp.dot(xh, weh_ref[...], preferred_element_type=f32)
           + be_ref[0][None, :])
    hs = y_nn + y_no * nz_ref[...]

    bf16 = jnp.bfloat16
    pde = pde_ref[...]
    hs_hi = lax.convert_element_type(hs, bf16)
    r1 = hs - lax.convert_element_type(hs_hi, f32)
    hs_md = lax.convert_element_type(r1, bf16)
    r2 = r1 - lax.convert_element_type(hs_md, f32)
    hs_lo = lax.convert_element_type(r2, bf16)
    hs_de = (jnp.dot(hs_hi, pde, preferred_element_type=f32)
             + jnp.dot(hs_md, pde, preferred_element_type=f32)
             + jnp.dot(hs_lo, pde, preferred_element_type=f32))
    ye_hi = lax.convert_element_type(y_e, bf16)
    ye_lo = lax.convert_element_type(
        y_e - lax.convert_element_type(ye_hi, f32), bf16)
    ye_de = (jnp.dot(ye_hi, pde, preferred_element_type=f32)
             + jnp.dot(ye_lo, pde, preferred_element_type=f32))

    ks = []
    for e in range(_E):
        b = lax.bitcast_convert_type(hs_de[:, e * _DC:(e + 1) * _DC],
                                     jnp.int32)
        o = b ^ (lax.shift_right_arithmetic(b, 31) & jnp.int32(0x7FFFFFFF))
        ks.append((o & jnp.int32(~7)) | jnp.int32(_E - 1 - e))
    m1k = ks[0]
    for e in range(1, _E):
        m1k = jnp.maximum(m1k, ks[e])
    k2s = [jnp.where(k == m1k, jnp.int32(-(2**31)), k) for k in ks]
    m2k = k2s[0]
    for e in range(1, _E):
        m2k = jnp.maximum(m2k, k2s[e])

    def _to_f32(v):
        return lax.bitcast_convert_type(
            v ^ (lax.shift_right_arithmetic(v, 31) & jnp.int32(0x7FFFFFFF)),
            jnp.float32)

    s = jnp.exp(_to_f32(m2k) - _to_f32(m1k))
    inv_z = 1.0 / (1.0 + s)
    g2 = s * inv_z
    acc = None
    for e in range(_E):
        ge = jnp.where(ks[e] == m1k, inv_z,
                       jnp.where(k2s[e] == m2k, g2, 0.0))
        term = ge * ye_de[:, e * _DC:(e + 1) * _DC]
        acc = term if acc is None else acc + term
    out_ref[...] = acc * (1.0 / _E)


def kernel(h, us, ue, u, W_non_noise, b_non_noise, W_noise, b_noise, W_E, b_E, W_r, b_r):
    f32 = jnp.float32

    hcat8 = jnp.broadcast_to(
        jnp.concatenate([h, us, ue], axis=-1).reshape(1, 5 * _D), (8, 5 * _D)
    )
    br8 = jnp.broadcast_to(b_r[None, :], (8, _D))
    bnn8 = jnp.broadcast_to(b_non_noise[None, :], (8, _D * _E))
    bno8 = jnp.broadcast_to(b_noise[None, :], (8, _D * _E))
    be8 = jnp.broadcast_to(b_E[None, :], (8, _D * _E))
    x2d = u.reshape(_S, 2 * _D)

    row2 = pl.BlockSpec((_KH, _BN), lambda c: (2, c))
    bspec = pl.BlockSpec((8, _BN), lambda c: (0, c))
    beff_nn, beff_no, beff_e = pl.pallas_call(
        _bias_kernel,
        grid=(_NC,),
        in_specs=[
            pl.BlockSpec((8, 5 * _D), lambda c: (0, 0)),
            pl.BlockSpec((5 * _D, _D), lambda c: (0, 0)),
            pl.BlockSpec((8, _D), lambda c: (0, 0)),
            row2, row2, row2,
            bspec, bspec, bspec,
        ],
        out_specs=[bspec, bspec, bspec],
        out_shape=[jax.ShapeDtypeStruct((8, _D * _E), f32)] * 3,
        scratch_shapes=[pltpu.VMEM((8, _D), f32)],
    )(hcat8, W_r, br8, W_non_noise, W_noise, W_E, bnn8, bno8, be8)

    row0 = pl.BlockSpec((_KH, _BN), lambda c, t: (0, c))
    row1 = pl.BlockSpec((_KH, _BN), lambda c, t: (1, c))
    bspec2 = pl.BlockSpec((8, _BN), lambda c, t: (0, c))
    out2d = pl.pallas_call(
        _main_kernel,
        grid=(_NC, _NT),
        in_specs=[
            pl.BlockSpec((_T, 2 * _D), lambda c, t: (t, 0)),
            row0, row1, row0, row1, row0, row1,
            bspec2, bspec2, bspec2,
            pl.BlockSpec((_T, _BN), lambda c, t: (t, c)),
            pl.BlockSpec((_BN, _BN), lambda c, t: (0, 0)),
        ],
        out_specs=pl.BlockSpec((_T, _DC), lambda c, t: (t, c)),
        out_shape=jax.ShapeDtypeStruct((_S, _D), f32),
    )(x2d, W_non_noise, W_non_noise, W_noise, W_noise, W_E, W_E,
      beff_nn, beff_no, beff_e, jnp.asarray(_NOISE),
      jnp.asarray(_PDE, jnp.bfloat16))

    return out2d.reshape(1, _S, _D)


# speedup vs baseline: 1.2658x; 1.0867x over previous
import jax
import jax.numpy as jnp
import numpy as np
import scipy.special as _sp
from jax import lax
from jax.experimental import pallas as pl
from jax.experimental.pallas import tpu as pltpu

_S = 2048
_D = 768
_E = 8
_KH = _D
_T = 512
_DC = 128
_NC = _D // _DC
_NT = _S // _T
_BN = _E * _DC

def _threefry2x32_np(k0, k1, x0, x1):
    rot_even = (13, 15, 26, 6)
    rot_odd = (17, 29, 16, 24)

    def _rotl(x, r):
        return ((x << np.uint32(r)) | (x >> np.uint32(32 - r))).astype(np.uint32)

    ks = (np.uint32(k0), np.uint32(k1),
          np.uint32(np.uint32(k0) ^ np.uint32(k1) ^ np.uint32(0x1BD11BDA)))
    x0 = (x0 + ks[0]).astype(np.uint32)
    x1 = (x1 + ks[1]).astype(np.uint32)
    for d in range(5):
        for r in rot_even if d % 2 == 0 else rot_odd:
            x0 = (x0 + x1).astype(np.uint32)
            x1 = _rotl(x1, r)
            x1 = (x1 ^ x0).astype(np.uint32)
        x0 = (x0 + ks[(d + 1) % 3]).astype(np.uint32)
        x1 = (x1 + ks[(d + 2) % 3] + np.uint32(d + 1)).astype(np.uint32)
    return x0, x1


def _noise_natural_np():
    n = _S * _D * _E
    i = np.arange(n, dtype=np.uint64)
    hi32 = (i >> np.uint64(32)).astype(np.uint32)
    lo32 = (i & np.uint64(0xFFFFFFFF)).astype(np.uint32)
    b0, b1 = _threefry2x32_np(0, 12345, hi32, lo32)
    bits = b0 ^ b1
    mant = (bits >> np.uint32(9)) | np.float32(1.0).view(np.uint32)
    f = mant.view(np.float32) - np.float32(1.0)
    lo_f = np.float32(np.nextafter(np.float32(-1.0), np.float32(0.0)))
    u = np.maximum(lo_f, (f * (np.float32(1.0) - lo_f) + lo_f).astype(np.float32))
    norm = (np.float32(np.sqrt(2.0))
            * _sp.erfinv(u.astype(np.float64)).astype(np.float32))
    return norm.astype(np.float32).reshape(_S, _D * _E)


_NOISE = _noise_natural_np()

_PDE = np.zeros((_BN, _BN), dtype=np.float32)
_i = np.arange(_BN)
_PDE[_i, (_i % _E) * _DC + _i // _E] = 1.0


def _bias_kernel(hcat_ref, wr_ref, br_ref, wnn_ref, wno_ref, we_ref,
                 bnn_ref, bno_ref, be_ref, onn_ref, ono_ref, oe_ref, r8):
    @pl.when(pl.program_id(0) == 0)
    def _():
        r8[...] = (
            jnp.dot(hcat_ref[...], wr_ref[...], preferred_element_type=jnp.float32)
            + br_ref[...]
        )

    r = r8[...]
    onn_ref[...] = jnp.dot(r, wnn_ref[...], preferred_element_type=jnp.float32) + bnn_ref[...]
    ono_ref[...] = jnp.dot(r, wno_ref[...], preferred_element_type=jnp.float32) + bno_ref[...]
    oe_ref[...] = jnp.dot(r, we_ref[...], preferred_element_type=jnp.float32) + be_ref[...]


def _main_kernel(x_ref, wnnl_ref, wnnh_ref, wnol_ref, wnoh_ref, wel_ref, weh_ref,
                 bnn_ref, bno_ref, be_ref, nz_ref, pde_ref, out_ref):
    f32 = jnp.float32
    xl = x_ref[:, :_KH]
    xh = x_ref[:, _KH:]
    y_nn = (jnp.dot(xl, wnnl_ref[...], preferred_element_type=f32)
            + jnp.dot(xh, wnnh_ref[...], preferred_element_type=f32)
            + bnn_ref[0][None, :])
    y_no = (jnp.dot(xl, wnol_ref[...], preferred_element_type=f32)
            + jnp.dot(xh, wnoh_ref[...], preferred_element_type=f32)
            + bno_ref[0][None, :])
    y_e = (jnp.dot(xl, wel_ref[...], preferred_element_type=f32)
           + jnp.dot(xh, weh_ref[...], preferred_element_type=f32)
           + be_ref[0][None, :])
    hs = y_nn + y_no * nz_ref[...]

    bf16 = jnp.bfloat16
    pde = pde_ref[...]
    hs_hi = lax.convert_element_type(hs, bf16)
    r1 = hs - lax.convert_element_type(hs_hi, f32)
    hs_md = lax.convert_element_type(r1, bf16)
    hs_de = (jnp.dot(hs_hi, pde, preferred_element_type=f32)
             + jnp.dot(hs_md, pde, preferred_element_type=f32))
    ye_hi = lax.convert_element_type(y_e, bf16)
    ye_lo = lax.convert_element_type(
        y_e - lax.convert_element_type(ye_hi, f32), bf16)
    ye_de = (jnp.dot(ye_hi, pde, preferred_element_type=f32)
             + jnp.dot(ye_lo, pde, preferred_element_type=f32))

    ks = []
    for e in range(_E):
        b = lax.bitcast_convert_type(hs_de[:, e * _DC:(e + 1) * _DC],
                                     jnp.int32)
        o = b ^ (lax.shift_right_arithmetic(b, 31) & jnp.int32(0x7FFFFFFF))
        ks.append((o & jnp.int32(~7)) | jnp.int32(_E - 1 - e))
    m1k = ks[0]
    for e in range(1, _E):
        m1k = jnp.maximum(m1k, ks[e])
    k2s = [jnp.where(k == m1k, jnp.int32(-(2**31)), k) for k in ks]
    m2k = k2s[0]
    for e in range(1, _E):
        m2k = jnp.maximum(m2k, k2s[e])

    def _to_f32(v):
        return lax.bitcast_convert_type(
            v ^ (lax.shift_right_arithmetic(v, 31) & jnp.int32(0x7FFFFFFF)),
            jnp.float32)

    s = jnp.exp(_to_f32(m2k) - _to_f32(m1k))
    inv_z = 1.0 / (1.0 + s)
    g2 = s * inv_z
    acc = None
    for e in range(_E):
        ge = jnp.where(ks[e] == m1k, inv_z,
                       jnp.where(k2s[e] == m2k, g2, 0.0))
        term = ge * ye_de[:, e * _DC:(e + 1) * _DC]
        acc = term if acc is None else acc + term
    out_ref[...] = acc * (1.0 / _E)


def kernel(h, us, ue, u, W_non_noise, b_non_noise, W_noise, b_noise, W_E, b_E, W_r, b_r):
    f32 = jnp.float32

    hcat8 = jnp.broadcast_to(
        jnp.concatenate([h, us, ue], axis=-1).reshape(1, 5 * _D), (8, 5 * _D)
    )
    br8 = jnp.broadcast_to(b_r[None, :], (8, _D))
    bnn8 = jnp.broadcast_to(b_non_noise[None, :], (8, _D * _E))
    bno8 = jnp.broadcast_to(b_noise[None, :], (8, _D * _E))
    be8 = jnp.broadcast_to(b_E[None, :], (8, _D * _E))
    x2d = u.reshape(_S, 2 * _D)

    row2 = pl.BlockSpec((_KH, _BN), lambda c: (2, c))
    bspec = pl.BlockSpec((8, _BN), lambda c: (0, c))
    beff_nn, beff_no, beff_e = pl.pallas_call(
        _bias_kernel,
        grid=(_NC,),
        in_specs=[
            pl.BlockSpec((8, 5 * _D), lambda c: (0, 0)),
            pl.BlockSpec((5 * _D, _D), lambda c: (0, 0)),
            pl.BlockSpec((8, _D), lambda c: (0, 0)),
            row2, row2, row2,
            bspec, bspec, bspec,
        ],
        out_specs=[bspec, bspec, bspec],
        out_shape=[jax.ShapeDtypeStruct((8, _D * _E), f32)] * 3,
        scratch_shapes=[pltpu.VMEM((8, _D), f32)],
    )(hcat8, W_r, br8, W_non_noise, W_noise, W_E, bnn8, bno8, be8)

    row0 = pl.BlockSpec((_KH, _BN), lambda c, t: (0, c))
    row1 = pl.BlockSpec((_KH, _BN), lambda c, t: (1, c))
    bspec2 = pl.BlockSpec((8, _BN), lambda c, t: (0, c))
    out2d = pl.pallas_call(
        _main_kernel,
        grid=(_NC, _NT),
        in_specs=[
            pl.BlockSpec((_T, 2 * _D), lambda c, t: (t, 0)),
            row0, row1, row0, row1, row0, row1,
            bspec2, bspec2, bspec2,
            pl.BlockSpec((_T, _BN), lambda c, t: (t, c)),
            pl.BlockSpec((_BN, _BN), lambda c, t: (0, 0)),
        ],
        out_specs=pl.BlockSpec((_T, _DC), lambda c, t: (t, c)),
        out_shape=jax.ShapeDtypeStruct((_S, _D), f32),
    )(x2d, W_non_noise, W_non_noise, W_noise, W_noise, W_E, W_E,
      beff_nn, beff_no, beff_e, jnp.asarray(_NOISE),
      jnp.asarray(_PDE, jnp.bfloat16))

    return out2d.reshape(1, _S, _D)
